# speedup vs baseline: 9.2311x; 9.2311x over previous
import functools

import jax
import jax.numpy as jnp
from jax.experimental import pallas as pl
from jax.experimental.pallas import tpu as pltpu

_TOPK = 32
_BISECT_ITERS = 26


def _dot(a, b):
    return jax.lax.dot_general(
        a, b, (((1,), (0,)), ((), ())),
        precision=jax.lax.Precision.HIGHEST,
        preferred_element_type=jnp.float32)


def _tile_kernel(x_ref, w1_ref, b1_ref, w2_ref, b2_ref, mem_ref,
                 d1_ref, db1_ref, d2_ref, db2_ref, out_ref, *, mem_size):
    x = x_ref[...]
    h = jnp.maximum(_dot(x, w1_ref[...]) + b1_ref[...], 0.0)
    z = _dot(h, w2_ref[...]) + b2_ref[...]
    zn = z / jnp.maximum(
        jnp.sqrt(jnp.sum(z * z, axis=1, keepdims=True)), 1e-12)

    mem = mem_ref[...]
    mn = mem / jnp.maximum(
        jnp.sqrt(jnp.sum(mem * mem, axis=1, keepdims=True)), 1e-12)
    sim = _dot(zn, mn.T)

    row_max = jnp.max(sim, axis=1, keepdims=True)
    hi0 = row_max + 1e-3
    lo0 = jnp.full_like(hi0, -1.001)

    def body(_, carry):
        lo, hi = carry
        mid = 0.5 * (lo + hi)
        cnt = jnp.sum(jnp.where(sim >= mid, 1.0, 0.0), axis=1,
                      keepdims=True)
        take = cnt >= float(_TOPK)
        return jnp.where(take, mid, lo), jnp.where(take, hi, mid)

    lo, hi = jax.lax.fori_loop(0, _BISECT_ITERS, body, (lo0, hi0))

    p = jnp.where(sim >= lo, jnp.exp(sim) - 1.0, 0.0)
    denom = float(mem_size) + jnp.sum(p, axis=1, keepdims=True)
    mem_sum = jnp.sum(mem, axis=0, keepdims=True)
    z_mem = (mem_sum + _dot(p, mem)) / denom

    d = jnp.maximum(_dot(z_mem, d1_ref[...]) + db1_ref[...], 0.0)
    logits = _dot(d, d2_ref[...]) + db2_ref[...]
    out_ref[...] = 1.0 / (1.0 + jnp.exp(-logits))


def kernel(x, enc_w1, enc_b1, enc_w2, enc_b2, memory,
           dec_w1, dec_b1, dec_w2, dec_b2):
    b, in_dim = x.shape
    mem_size, embed_dim = memory.shape
    hid = enc_w1.shape[0]

    bt = 128
    assert b % bt == 0
    grid = (b // bt,)

    full = lambda shape: pl.BlockSpec(shape, lambda i: (0, 0))

    fn = pl.pallas_call(
        functools.partial(_tile_kernel, mem_size=mem_size),
        grid=grid,
        in_specs=[
            pl.BlockSpec((bt, in_dim), lambda i: (i, 0)),
            full((in_dim, hid)),
            full((1, hid)),
            full((hid, embed_dim)),
            full((1, embed_dim)),
            full((mem_size, embed_dim)),
            full((embed_dim, hid)),
            full((1, hid)),
            full((hid, in_dim)),
            full((1, in_dim)),
        ],
        out_specs=pl.BlockSpec((bt, in_dim), lambda i: (i, 0)),
        out_shape=jax.ShapeDtypeStruct((b, in_dim), jnp.float32),
        compiler_params=pltpu.CompilerParams(
            dimension_semantics=("arbitrary",)),
    )
    return fn(x, enc_w1.T, enc_b1.reshape(1, -1), enc_w2.T,
              enc_b2.reshape(1, -1), memory, dec_w1.T,
              dec_b1.reshape(1, -1), dec_w2.T, dec_b2.reshape(1, -1))


# speedup vs baseline: 9.2973x; 1.0072x over previous
import functools

import jax
import jax.numpy as jnp
from jax.experimental import pallas as pl
from jax.experimental.pallas import tpu as pltpu

_TOPK = 32
_BISECT_ITERS = 40


def _dot(a, b):
    return jax.lax.dot_general(
        a, b, (((1,), (0,)), ((), ())),
        precision=jax.lax.Precision.HIGHEST,
        preferred_element_type=jnp.float32)


def _tile_kernel(x_ref, w1_ref, b1_ref, w2_ref, b2_ref, mem_ref,
                 d1_ref, db1_ref, d2_ref, db2_ref, out_ref,
                 mn_ref, msum_ref, *, mem_size):
    @pl.when(pl.program_id(0) == 0)
    def _():
        mem = mem_ref[...]
        mn_ref[...] = mem / jnp.maximum(
            jnp.sqrt(jnp.sum(mem * mem, axis=1, keepdims=True)), 1e-12)
        msum_ref[...] = jnp.sum(mem, axis=0, keepdims=True)

    x = x_ref[...]
    h = jnp.maximum(_dot(x, w1_ref[...]) + b1_ref[...], 0.0)
    z = _dot(h, w2_ref[...]) + b2_ref[...]
    zn = z / jnp.maximum(
        jnp.sqrt(jnp.sum(z * z, axis=1, keepdims=True)), 1e-12)

    sim = _dot(zn, mn_ref[...].T)
    bt = sim.shape[0]

    def count_ge(t):
        return jnp.sum(jnp.where(sim >= t, 1.0, 0.0), axis=1,
                       keepdims=True)

    row_max = jnp.max(sim, axis=1, keepdims=True)
    hi0 = row_max + 1e-3
    lo0 = jnp.full_like(hi0, -1.001)
    kf = float(_TOPK)

    def cond(carry):
        i, lo, hi, cnt_lo, cnt_hi = carry
        return jnp.logical_and(i < _BISECT_ITERS,
                               jnp.sum(cnt_lo) > kf * bt)

    def body(carry):
        i, lo, hi, cnt_lo, cnt_hi = carry
        width = hi - lo
        frac = jnp.clip((cnt_lo - kf) / (cnt_lo - cnt_hi), 0.02, 0.98)
        mid = jnp.where(i % 2 == 0, lo + frac * width, lo + 0.5 * width)
        cnt = count_ge(mid)
        take = cnt >= kf
        lo = jnp.where(take, mid, lo)
        cnt_lo = jnp.where(take, cnt, cnt_lo)
        hi = jnp.where(take, hi, mid)
        cnt_hi = jnp.where(take, cnt_hi, cnt)
        return i + 1, lo, hi, cnt_lo, cnt_hi

    _, lo, hi, cnt_lo, cnt_hi = jax.lax.while_loop(
        cond, body,
        (0, lo0, hi0, jnp.full_like(lo0, float(mem_size)),
         jnp.zeros_like(lo0)))

    p = jnp.where(sim >= lo, jnp.exp(sim) - 1.0, 0.0)
    denom = float(mem_size) + jnp.sum(p, axis=1, keepdims=True)
    z_mem = (msum_ref[...] + _dot(p, mem_ref[...])) / denom

    d = jnp.maximum(_dot(z_mem, d1_ref[...]) + db1_ref[...], 0.0)
    logits = _dot(d, d2_ref[...]) + db2_ref[...]
    out_ref[...] = 1.0 / (1.0 + jnp.exp(-logits))


def kernel(x, enc_w1, enc_b1, enc_w2, enc_b2, memory,
           dec_w1, dec_b1, dec_w2, dec_b2):
    b, in_dim = x.shape
    mem_size, embed_dim = memory.shape
    hid = enc_w1.shape[0]

    bt = 128
    assert b % bt == 0
    grid = (b // bt,)

    full = lambda shape: pl.BlockSpec(shape, lambda i: (0, 0))

    fn = pl.pallas_call(
        functools.partial(_tile_kernel, mem_size=mem_size),
        grid=grid,
        in_specs=[
            pl.BlockSpec((bt, in_dim), lambda i: (i, 0)),
            full((in_dim, hid)),
            full((1, hid)),
            full((hid, embed_dim)),
            full((1, embed_dim)),
            full((mem_size, embed_dim)),
            full((embed_dim, hid)),
            full((1, hid)),
            full((hid, in_dim)),
            full((1, in_dim)),
        ],
        out_specs=pl.BlockSpec((bt, in_dim), lambda i: (i, 0)),
        out_shape=jax.ShapeDtypeStruct((b, in_dim), jnp.float32),
        scratch_shapes=[
            pltpu.VMEM((mem_size, embed_dim), jnp.float32),
            pltpu.VMEM((1, embed_dim), jnp.float32),
        ],
        compiler_params=pltpu.CompilerParams(
            dimension_semantics=("arbitrary",)),
    )
    return fn(x, enc_w1.T, enc_b1.reshape(1, -1), enc_w2.T,
              enc_b2.reshape(1, -1), memory, dec_w1.T,
              dec_b1.reshape(1, -1), dec_w2.T, dec_b2.reshape(1, -1))


# speedup vs baseline: 10.7778x; 1.1592x over previous
import functools

import jax
import jax.numpy as jnp
from jax.experimental import pallas as pl
from jax.experimental.pallas import tpu as pltpu

_TOPK = 32
_BISECT_ITERS = 64


def _dot(a, b):
    return jax.lax.dot_general(
        a, b, (((1,), (0,)), ((), ())),
        precision=jax.lax.Precision.HIGHEST,
        preferred_element_type=jnp.float32)


def _tile_kernel(x_ref, w1_ref, b1_ref, w2_ref, b2_ref, mem_ref,
                 d1_ref, db1_ref, d2_ref, db2_ref, out_ref,
                 mn_ref, msum_ref, *, mem_size):
    @pl.when(pl.program_id(0) == 0)
    def _():
        mem = mem_ref[...]
        mn_ref[...] = mem / jnp.maximum(
            jnp.sqrt(jnp.sum(mem * mem, axis=1, keepdims=True)), 1e-12)
        msum_ref[...] = jnp.sum(mem, axis=0, keepdims=True)

    x = x_ref[...]
    h = jnp.maximum(_dot(x, w1_ref[...]) + b1_ref[...], 0.0)
    z = _dot(h, w2_ref[...]) + b2_ref[...]
    zn = z / jnp.maximum(
        jnp.sqrt(jnp.sum(z * z, axis=1, keepdims=True)), 1e-12)

    sim = _dot(zn, mn_ref[...].T)
    bt = sim.shape[0]

    def count_ge(t):
        return jnp.sum(jnp.where(sim >= t, 1.0, 0.0), axis=1,
                       keepdims=True)

    row_max = jnp.max(sim, axis=1, keepdims=True)
    hi0 = row_max + 1e-3
    lo0 = jnp.full_like(hi0, -1.001)
    kf = float(_TOPK)
    n = float(sim.shape[1])
    mu = jnp.sum(sim, axis=1, keepdims=True) / n
    sig = jnp.sqrt(jnp.maximum(
        jnp.sum(sim * sim, axis=1, keepdims=True) / n - mu * mu, 0.0))

    def cond(carry):
        i, lo, hi, cnt_lo, cnt_hi = carry
        return jnp.logical_and(i < _BISECT_ITERS,
                               jnp.sum(cnt_lo) > kf * bt)

    def body(carry):
        i, lo, hi, cnt_lo, cnt_hi = carry
        width = hi - lo
        fl = jnp.log(cnt_lo + 0.5)
        fh = jnp.log(cnt_hi + 0.5)
        ft = jnp.log(kf + 0.5)
        frac = jnp.clip((fl - ft) / (fl - fh), 0.02, 0.98)
        probe = jnp.where(i == 0, mu + 2.3944 * sig,
                          jnp.where(cnt_lo > 0.5 * n, mu + 2.0 * sig,
                                    lo + frac * width))
        mid = jnp.clip(probe, lo + 0.02 * width, hi - 0.02 * width)
        mid = jnp.where(i % 3 == 2, lo + 0.5 * width, mid)
        cnt = count_ge(mid)
        take = cnt >= kf
        lo = jnp.where(take, mid, lo)
        cnt_lo = jnp.where(take, cnt, cnt_lo)
        hi = jnp.where(take, hi, mid)
        cnt_hi = jnp.where(take, cnt_hi, cnt)
        return i + 1, lo, hi, cnt_lo, cnt_hi

    _, lo, hi, cnt_lo, cnt_hi = jax.lax.while_loop(
        cond, body,
        (0, lo0, hi0, jnp.full_like(lo0, float(mem_size)),
         jnp.zeros_like(lo0)))

    p = jnp.where(sim >= lo, jnp.exp(sim) - 1.0, 0.0)
    denom = float(mem_size) + jnp.sum(p, axis=1, keepdims=True)
    z_mem = (msum_ref[...] + _dot(p, mem_ref[...])) / denom

    d = jnp.maximum(_dot(z_mem, d1_ref[...]) + db1_ref[...], 0.0)
    logits = _dot(d, d2_ref[...]) + db2_ref[...]
    out_ref[...] = 1.0 / (1.0 + jnp.exp(-logits))


def kernel(x, enc_w1, enc_b1, enc_w2, enc_b2, memory,
           dec_w1, dec_b1, dec_w2, dec_b2):
    b, in_dim = x.shape
    mem_size, embed_dim = memory.shape
    hid = enc_w1.shape[0]

    bt = 128
    assert b % bt == 0
    grid = (b // bt,)

    full = lambda shape: pl.BlockSpec(shape, lambda i: (0, 0))

    fn = pl.pallas_call(
        functools.partial(_tile_kernel, mem_size=mem_size),
        grid=grid,
        in_specs=[
            pl.BlockSpec((bt, in_dim), lambda i: (i, 0)),
            full((in_dim, hid)),
            full((1, hid)),
            full((hid, embed_dim)),
            full((1, embed_dim)),
            full((mem_size, embed_dim)),
            full((embed_dim, hid)),
            full((1, hid)),
            full((hid, in_dim)),
            full((1, in_dim)),
        ],
        out_specs=pl.BlockSpec((bt, in_dim), lambda i: (i, 0)),
        out_shape=jax.ShapeDtypeStruct((b, in_dim), jnp.float32),
        scratch_shapes=[
            pltpu.VMEM((mem_size, embed_dim), jnp.float32),
            pltpu.VMEM((1, embed_dim), jnp.float32),
        ],
        compiler_params=pltpu.CompilerParams(
            dimension_semantics=("arbitrary",)),
    )
    return fn(x, enc_w1.T, enc_b1.reshape(1, -1), enc_w2.T,
              enc_b2.reshape(1, -1), memory, dec_w1.T,
              dec_b1.reshape(1, -1), dec_w2.T, dec_b2.reshape(1, -1))
